# baseline (speedup 1.0000x reference)
import functools

import jax
import jax.numpy as jnp
from jax import lax
from jax.experimental import pallas as pl
from jax.experimental.pallas import tpu as pltpu
from jax.experimental.pallas import tpu_sc as plsc

BS = 256


RB = 256


def _routing_body(x_ref, c_ref, pos_ref, b2e_ref, best_s, rank_s, carry_s,
                  *, Bn, En, nblk, nsteps):
    t = pl.program_id(0)

    @pl.when(t == 0)
    def _():
        carry_s[...] = jnp.zeros_like(carry_s)

    @pl.when(t < nsteps)
    def _():
        xb = x_ref[...]
        best_d = jnp.full((RB, 1), jnp.inf, jnp.float32)
        best_i = jnp.zeros((RB, 1), jnp.int32)
        for e in range(En):
            diff = xb - c_ref[e:e + 1, :]
            d2 = jnp.sum(diff * diff, axis=1, keepdims=True)
            upd = d2 < best_d
            best_i = jnp.where(upd, e, best_i)
            best_d = jnp.where(upd, d2, best_d)

        lane = lax.broadcasted_iota(jnp.int32, (RB, En), 1)
        oh = (best_i == lane).astype(jnp.int32)
        cs = oh
        s = 1
        while s < RB:
            shifted = jnp.concatenate(
                [jnp.zeros((s, En), jnp.int32), cs[:RB - s, :]], axis=0)
            cs = cs + shifted
            s *= 2
        carry = carry_s[...]
        rank1 = jnp.sum(oh * (carry + cs - oh), axis=1, keepdims=True)
        best_s[pl.ds(t * RB, RB), :] = best_i
        rank_s[pl.ds(t * RB, RB), :] = rank1
        carry_s[...] = carry + cs[RB - 1:RB, :]

    @pl.when(t == nsteps)
    def _():
        counts = carry_s[...]
        padded = ((counts + (BS - 1)) // BS) * BS

        pcb = jnp.broadcast_to(padded.astype(jnp.float32), (En, En))
        r = lax.broadcasted_iota(jnp.int32, (En, En), 0)
        c = lax.broadcasted_iota(jnp.int32, (En, En), 1)
        tri = (r < c).astype(jnp.float32)
        offs = lax.dot(pcb, tri,
                       preferred_element_type=jnp.float32)[0:1, :]
        offs = offs.astype(jnp.int32)

        lane = lax.broadcasted_iota(jnp.int32, (Bn, En), 1)
        oh2 = (best_s[...] == lane).astype(jnp.int32)
        pos_ref[...] = jnp.sum(oh2 * offs, axis=1) + rank_s[...][:, 0]

        kstart = lax.broadcasted_iota(jnp.int32, (nblk, En), 0) * BS
        ge = (kstart >= jnp.broadcast_to(offs, (nblk, En))).astype(jnp.int32)
        b2e = jnp.sum(ge, axis=1) - 1
        total = jnp.sum(padded)
        kcol = lax.broadcasted_iota(jnp.int32, (nblk,), 0) * BS
        b2e_ref[...] = jnp.where(kcol < total, b2e, -1)


def _ae_body(b2e_s, xs_ref, c_ref, we_ref, be_ref, wd_ref, bd_ref, out_ref,
             *, half):
    k = pl.program_id(0) * half + pl.program_id(1)
    e = b2e_s[k]

    @pl.when(e >= 0)
    def _():
        xc = xs_ref[...] - c_ref[0]
        h = jnp.tanh(
            lax.dot(xc, we_ref[0], preferred_element_type=jnp.float32)
            + be_ref[0])
        out_ref[...] = (
            lax.dot(h, wd_ref[0], preferred_element_type=jnp.float32)
            + bd_ref[0])


def kernel(x, centers, W_e, b_e, W_d, b_d):
    B, D = x.shape
    E, _, H = W_e.shape
    nblk = B // BS + E
    tot = nblk * BS
    f32 = jnp.float32

    nsteps = B // RB
    pos, b2e = pl.pallas_call(
        functools.partial(_routing_body, Bn=B, En=E, nblk=nblk,
                          nsteps=nsteps),
        grid=(nsteps + 1,),
        in_specs=[
            pl.BlockSpec((RB, D), lambda t: (jnp.minimum(t, nsteps - 1), 0)),
            pl.BlockSpec((E, D), lambda t: (0, 0)),
        ],
        out_specs=(pl.BlockSpec((B,), lambda t: (0,)),
                   pl.BlockSpec((nblk,), lambda t: (0,))),
        out_shape=(jax.ShapeDtypeStruct((B,), jnp.int32),
                   jax.ShapeDtypeStruct((nblk,), jnp.int32)),
        scratch_shapes=[pltpu.VMEM((B, 1), jnp.int32),
                        pltpu.VMEM((B, 1), jnp.int32),
                        pltpu.VMEM((1, E), jnp.int32)],
        compiler_params=pltpu.CompilerParams(
            dimension_semantics=("arbitrary",)),
    )(x, centers)

    mesh = plsc.VectorSubcoreMesh(core_axis_name="c", subcore_axis_name="s",
                                  num_cores=2, num_subcores=16)
    nw = 32
    rpw = B // nw

    @functools.partial(
        pl.kernel, mesh=mesh,
        out_type=jax.ShapeDtypeStruct((tot, D), f32),
        scratch_types=[pltpu.VMEM((rpw,), jnp.int32),
                       pltpu.VMEM((rpw, D), f32),
                       pltpu.SemaphoreType.DMA],
    )
    def dispatch(x_hbm, pos_hbm, xs_hbm, idx_v, rows_v, sem):
        wid = lax.axis_index("s") * 2 + lax.axis_index("c")
        base = wid * rpw
        pltpu.sync_copy(pos_hbm.at[pl.ds(base, rpw)], idx_v)
        pltpu.sync_copy(x_hbm.at[pl.ds(base, rpw)], rows_v)
        pltpu.async_copy(rows_v, xs_hbm.at[idx_v], sem).wait()

    xs = dispatch(x, pos)

    half = nblk // 2

    def kid(c, j, b2e_ref):
        return c * half + j

    def e_of(c, j, b2e_ref):
        k = kid(c, j, b2e_ref)
        return jnp.where(b2e_ref[k] < 0, E - 1, b2e_ref[k])

    def x_of(c, j, b2e_ref):
        k = kid(c, j, b2e_ref)
        return jnp.where(b2e_ref[k] < 0, 0, k)

    def y_of(c, j, b2e_ref):
        k = kid(c, j, b2e_ref)
        return jnp.where(b2e_ref[k] < 0, nblk - 1, k)

    grid_spec = pltpu.PrefetchScalarGridSpec(
        num_scalar_prefetch=1,
        grid=(2, half),
        in_specs=[
            pl.BlockSpec((BS, D), lambda c, j, b: (x_of(c, j, b), 0)),
            pl.BlockSpec((1, 1, D), lambda c, j, b: (e_of(c, j, b), 0, 0)),
            pl.BlockSpec((1, D, H), lambda c, j, b: (e_of(c, j, b), 0, 0)),
            pl.BlockSpec((1, 1, H), lambda c, j, b: (e_of(c, j, b), 0, 0)),
            pl.BlockSpec((1, H, D), lambda c, j, b: (e_of(c, j, b), 0, 0)),
            pl.BlockSpec((1, 1, D), lambda c, j, b: (e_of(c, j, b), 0, 0)),
        ],
        out_specs=pl.BlockSpec((BS, D), lambda c, j, b: (y_of(c, j, b), 0)),
    )
    ys = pl.pallas_call(
        functools.partial(_ae_body, half=half),
        grid_spec=grid_spec,
        out_shape=jax.ShapeDtypeStruct((tot, D), f32),
        compiler_params=pltpu.CompilerParams(
            dimension_semantics=("parallel", "arbitrary")),
    )(b2e, xs, centers.reshape(E, 1, D), W_e, b_e.reshape(E, 1, H),
      W_d, b_d.reshape(E, 1, D))

    @functools.partial(
        pl.kernel, mesh=mesh,
        out_type=jax.ShapeDtypeStruct((B, D), f32),
        scratch_types=[pltpu.VMEM((rpw,), jnp.int32),
                       pltpu.VMEM((rpw, D), f32),
                       pltpu.SemaphoreType.DMA],
    )
    def combine(ys_hbm, pos_hbm, out_hbm, idx_v, rows_v, sem):
        wid = lax.axis_index("s") * 2 + lax.axis_index("c")
        base = wid * rpw
        pltpu.sync_copy(pos_hbm.at[pl.ds(base, rpw)], idx_v)
        pltpu.async_copy(ys_hbm.at[idx_v], rows_v, sem).wait()
        pltpu.sync_copy(rows_v, out_hbm.at[pl.ds(base, rpw)])

    return combine(ys, pos)


# speedup vs baseline: 1.0178x; 1.0178x over previous
import functools

import jax
import jax.numpy as jnp
from jax import lax
from jax.experimental import pallas as pl
from jax.experimental.pallas import tpu as pltpu
from jax.experimental.pallas import tpu_sc as plsc

BS = 256


RB = 512


def _routing_body(x_ref, c_ref, pos_ref, b2e_ref, best_s, rank_s, carry_s,
                  *, Bn, En, nblk, nsteps):
    t = pl.program_id(0)

    @pl.when(t == 0)
    def _():
        carry_s[...] = jnp.zeros_like(carry_s)

    @pl.when(t < nsteps)
    def _():
        xb = x_ref[...]
        best_d = jnp.full((RB, 1), jnp.inf, jnp.float32)
        best_i = jnp.zeros((RB, 1), jnp.int32)
        for e in range(En):
            diff = xb - c_ref[e:e + 1, :]
            d2 = jnp.sum(diff * diff, axis=1, keepdims=True)
            upd = d2 < best_d
            best_i = jnp.where(upd, e, best_i)
            best_d = jnp.where(upd, d2, best_d)

        lane = lax.broadcasted_iota(jnp.int32, (RB, En), 1)
        oh = (best_i == lane).astype(jnp.int32)
        cs = oh
        s = 1
        while s < RB:
            shifted = jnp.concatenate(
                [jnp.zeros((s, En), jnp.int32), cs[:RB - s, :]], axis=0)
            cs = cs + shifted
            s *= 2
        carry = carry_s[...]
        rank1 = jnp.sum(oh * (carry + cs - oh), axis=1, keepdims=True)
        best_s[pl.ds(t * RB, RB), :] = best_i
        rank_s[pl.ds(t * RB, RB), :] = rank1
        carry_s[...] = carry + cs[RB - 1:RB, :]

    @pl.when(t == nsteps)
    def _():
        counts = carry_s[...]
        padded = ((counts + (BS - 1)) // BS) * BS

        pcb = jnp.broadcast_to(padded.astype(jnp.float32), (En, En))
        r = lax.broadcasted_iota(jnp.int32, (En, En), 0)
        c = lax.broadcasted_iota(jnp.int32, (En, En), 1)
        tri = (r < c).astype(jnp.float32)
        offs = lax.dot(pcb, tri,
                       preferred_element_type=jnp.float32)[0:1, :]
        offs = offs.astype(jnp.int32)

        lane = lax.broadcasted_iota(jnp.int32, (Bn, En), 1)
        oh2 = (best_s[...] == lane).astype(jnp.int32)
        pos_ref[...] = jnp.sum(oh2 * offs, axis=1) + rank_s[...][:, 0]

        kstart = lax.broadcasted_iota(jnp.int32, (nblk, En), 0) * BS
        ge = (kstart >= jnp.broadcast_to(offs, (nblk, En))).astype(jnp.int32)
        b2e = jnp.sum(ge, axis=1) - 1
        total = jnp.sum(padded)
        kcol = lax.broadcasted_iota(jnp.int32, (nblk,), 0) * BS
        b2e_ref[...] = jnp.where(kcol < total, b2e, -1)


def _ae_body(b2e_s, xs_ref, c_ref, we_ref, be_ref, wd_ref, bd_ref, out_ref,
             *, half):
    k = pl.program_id(0) * half + pl.program_id(1)
    e = b2e_s[k]

    @pl.when(e >= 0)
    def _():
        xc = xs_ref[...] - c_ref[0]
        h = jnp.tanh(
            lax.dot(xc, we_ref[0], preferred_element_type=jnp.float32)
            + be_ref[0])
        out_ref[...] = (
            lax.dot(h, wd_ref[0], preferred_element_type=jnp.float32)
            + bd_ref[0])


def kernel(x, centers, W_e, b_e, W_d, b_d):
    B, D = x.shape
    E, _, H = W_e.shape
    nblk = B // BS + E
    tot = nblk * BS
    f32 = jnp.float32

    nsteps = B // RB
    pos, b2e = pl.pallas_call(
        functools.partial(_routing_body, Bn=B, En=E, nblk=nblk,
                          nsteps=nsteps),
        grid=(nsteps + 1,),
        in_specs=[
            pl.BlockSpec((RB, D), lambda t: (jnp.minimum(t, nsteps - 1), 0)),
            pl.BlockSpec((E, D), lambda t: (0, 0)),
        ],
        out_specs=(pl.BlockSpec((B,), lambda t: (0,)),
                   pl.BlockSpec((nblk,), lambda t: (0,))),
        out_shape=(jax.ShapeDtypeStruct((B,), jnp.int32),
                   jax.ShapeDtypeStruct((nblk,), jnp.int32)),
        scratch_shapes=[pltpu.VMEM((B, 1), jnp.int32),
                        pltpu.VMEM((B, 1), jnp.int32),
                        pltpu.VMEM((1, E), jnp.int32)],
        compiler_params=pltpu.CompilerParams(
            dimension_semantics=("arbitrary",)),
    )(x, centers)

    mesh = plsc.VectorSubcoreMesh(core_axis_name="c", subcore_axis_name="s",
                                  num_cores=2, num_subcores=16)
    nw = 32
    rpw = B // nw

    @functools.partial(
        pl.kernel, mesh=mesh,
        out_type=jax.ShapeDtypeStruct((tot, D), f32),
        scratch_types=[pltpu.VMEM((rpw,), jnp.int32),
                       pltpu.VMEM((rpw, D), f32),
                       pltpu.SemaphoreType.DMA],
    )
    def dispatch(x_hbm, pos_hbm, xs_hbm, idx_v, rows_v, sem):
        wid = lax.axis_index("s") * 2 + lax.axis_index("c")
        base = wid * rpw
        pltpu.sync_copy(pos_hbm.at[pl.ds(base, rpw)], idx_v)
        pltpu.sync_copy(x_hbm.at[pl.ds(base, rpw)], rows_v)
        pltpu.async_copy(rows_v, xs_hbm.at[idx_v], sem).wait()

    xs = dispatch(x, pos)

    half = nblk // 2

    def kid(c, j, b2e_ref):
        return c * half + j

    def e_of(c, j, b2e_ref):
        k = kid(c, j, b2e_ref)
        return jnp.where(b2e_ref[k] < 0, E - 1, b2e_ref[k])

    def x_of(c, j, b2e_ref):
        k = kid(c, j, b2e_ref)
        return jnp.where(b2e_ref[k] < 0, 0, k)

    def y_of(c, j, b2e_ref):
        k = kid(c, j, b2e_ref)
        return jnp.where(b2e_ref[k] < 0, nblk - 1, k)

    grid_spec = pltpu.PrefetchScalarGridSpec(
        num_scalar_prefetch=1,
        grid=(2, half),
        in_specs=[
            pl.BlockSpec((BS, D), lambda c, j, b: (x_of(c, j, b), 0)),
            pl.BlockSpec((1, 1, D), lambda c, j, b: (e_of(c, j, b), 0, 0)),
            pl.BlockSpec((1, D, H), lambda c, j, b: (e_of(c, j, b), 0, 0)),
            pl.BlockSpec((1, 1, H), lambda c, j, b: (e_of(c, j, b), 0, 0)),
            pl.BlockSpec((1, H, D), lambda c, j, b: (e_of(c, j, b), 0, 0)),
            pl.BlockSpec((1, 1, D), lambda c, j, b: (e_of(c, j, b), 0, 0)),
        ],
        out_specs=pl.BlockSpec((BS, D), lambda c, j, b: (y_of(c, j, b), 0)),
    )
    ys = pl.pallas_call(
        functools.partial(_ae_body, half=half),
        grid_spec=grid_spec,
        out_shape=jax.ShapeDtypeStruct((tot, D), f32),
        compiler_params=pltpu.CompilerParams(
            dimension_semantics=("parallel", "arbitrary")),
    )(b2e, xs, centers.reshape(E, 1, D), W_e, b_e.reshape(E, 1, H),
      W_d, b_d.reshape(E, 1, D))

    @functools.partial(
        pl.kernel, mesh=mesh,
        out_type=jax.ShapeDtypeStruct((B, D), f32),
        scratch_types=[pltpu.VMEM((rpw,), jnp.int32),
                       pltpu.VMEM((rpw, D), f32),
                       pltpu.SemaphoreType.DMA],
    )
    def combine(ys_hbm, pos_hbm, out_hbm, idx_v, rows_v, sem):
        wid = lax.axis_index("s") * 2 + lax.axis_index("c")
        base = wid * rpw
        pltpu.sync_copy(pos_hbm.at[pl.ds(base, rpw)], idx_v)
        pltpu.async_copy(ys_hbm.at[idx_v], rows_v, sem).wait()
        pltpu.sync_copy(rows_v, out_hbm.at[pl.ds(base, rpw)])

    return combine(ys, pos)
